# baseline (speedup 1.0000x reference)
import functools

import jax
import jax.numpy as jnp
from jax import lax
from jax.experimental import pallas as pl
from jax.experimental.pallas import tpu as pltpu
from jax.experimental.pallas import tpu_sc as plsc

B = 16384
D = 64
N_NEG = 4
NC = 2
NS = 16
L = 16
NW = NC * NS
R = B // NW
C = 64
NCHUNK = R // C
NBUF = 2
UNROLL = 4


def _bpr_body(u_hbm, i_hbm, jst_hbm, tu_hbm, ti_hbm, out_hbm,
              uidx, iidx, jidx, uhalf, ihalf, jhalf, *flat_scratch):
    wid = lax.axis_index("s") * NC + lax.axis_index("c")
    base = wid * R
    per = len(flat_scratch) // NBUF
    scratch = [flat_scratch[p * per:(p + 1) * per] for p in range(NBUF)]
    lanes = lax.iota(jnp.int32, L)
    zero = jnp.zeros((L,), jnp.float32)
    one = jnp.full((L,), 1, jnp.int32)

    pltpu.sync_copy(u_hbm.at[pl.ds(base, R)], uidx)
    pltpu.sync_copy(i_hbm.at[pl.ds(base, R)], iidx)
    pltpu.sync_copy(jst_hbm.at[:, pl.ds(base, R)], jidx)
    for g in range(R // L):
        sl = pl.ds(g * L, L)
        uhalf[sl] = jax.lax.shift_right_logical(uidx[sl], 1)
        ihalf[sl] = jax.lax.shift_right_logical(iidx[sl], 1)
        for jn in range(N_NEG):
            jhalf[jn, sl] = jax.lax.shift_right_logical(jidx[jn, sl], 1)

    def fire(c):
        ubuf, ibuf, jbuf, _, sem = scratch[c % NBUF]
        sl = pl.ds(c * C, C)
        cps = [
            pltpu.async_copy(tu_hbm.at[uhalf.at[sl]], ubuf, sem),
            pltpu.async_copy(ti_hbm.at[ihalf.at[sl]], ibuf, sem),
        ]
        for jn in range(N_NEG):
            cps.append(
                pltpu.async_copy(ti_hbm.at[jhalf.at[jn, sl]], jbuf.at[jn], sem))
        return cps

    def compute(c):
        ubuf, ibuf, jbuf, obuf, _ = scratch[c % NBUF]
        for g in range(C // L):
            sl = pl.ds(c * C + g * L, L)
            rows = g * L + lanes
            pu = (uidx[sl] & one) * D
            pi = (iidx[sl] & one) * D
            pj = [(jidx[jn, sl] & one) * D for jn in range(N_NEG)]

            def dbody(it, carry, rows=rows, pu=pu, pi=pi, pj=pj):
                pos, neg = carry
                for q in range(UNROLL):
                    d = it * UNROLL + q
                    uv = plsc.load_gather(ubuf, [rows, pu + d])
                    iv = plsc.load_gather(ibuf, [rows, pi + d])
                    jsum = zero
                    for jn in range(N_NEG):
                        jv = plsc.load_gather(
                            jbuf,
                            [jnp.full((L,), jn, jnp.int32), rows, pj[jn] + d])
                        jsum = jsum + jv
                    pos = pos + uv * iv
                    neg = neg + uv * jsum
                return pos, neg

            pos, neg = lax.fori_loop(0, D // UNROLL, dbody, (zero, zero))
            x = pos - neg * (1.0 / N_NEG)
            obuf[pl.ds(g * L, L)] = 1.0 / (1.0 + jnp.exp(-x))
        pltpu.sync_copy(obuf, out_hbm.at[pl.ds(base + c * C, C)])

    pending = {0: fire(0)}
    for c in range(NCHUNK):
        if c + 1 < NCHUNK:
            pending[(c + 1) % NBUF] = fire(c + 1)
        for cp in pending.pop(c % NBUF):
            cp.wait()
        compute(c)


def _buf_scratch():
    return [
        pltpu.VMEM((C, 2 * D), jnp.float32),
        pltpu.VMEM((C, 2 * D), jnp.float32),
        pltpu.VMEM((N_NEG, C, 2 * D), jnp.float32),
        pltpu.VMEM((C,), jnp.float32),
        pltpu.SemaphoreType.DMA,
    ]


_bpr = functools.partial(
    pl.kernel,
    mesh=plsc.VectorSubcoreMesh(core_axis_name="c", subcore_axis_name="s"),
    compiler_params=pltpu.CompilerParams(needs_layout_passes=False),
    out_type=jax.ShapeDtypeStruct((B,), jnp.float32),
    scratch_types=[
        pltpu.VMEM((R,), jnp.int32),
        pltpu.VMEM((R,), jnp.int32),
        pltpu.VMEM((N_NEG, R), jnp.int32),
        pltpu.VMEM((R,), jnp.int32),
        pltpu.VMEM((R,), jnp.int32),
        pltpu.VMEM((N_NEG, R), jnp.int32),
    ] + _buf_scratch() + _buf_scratch(),
)(_bpr_body)


def kernel(u, i, js, emb_user, emb_item):
    js_t = js.T
    tu = emb_user.reshape(emb_user.shape[0] // 2, 2 * D)
    ti = emb_item.reshape(emb_item.shape[0] // 2, 2 * D)
    return _bpr(u.astype(jnp.int32), i.astype(jnp.int32),
                js_t.astype(jnp.int32), tu, ti)


# speedup vs baseline: 1.2452x; 1.2452x over previous
import functools

import jax
import jax.numpy as jnp
from jax import lax
from jax.experimental import pallas as pl
from jax.experimental.pallas import tpu as pltpu
from jax.experimental.pallas import tpu_sc as plsc

B = 16384
D = 64
N_NEG = 4
NC = 2
NS = 16
L = 16
NW = NC * NS
R = B // NW
C = 16
NCHUNK = R // C
NBUF = 2
UNROLL = 4


def _bpr_body(u_hbm, i_hbm, jst_hbm, tu_hbm, ti_hbm, out_hbm,
              uidx, iidx, jidx, *flat_scratch):
    wid = lax.axis_index("s") * NC + lax.axis_index("c")
    base = wid * R
    scratch = [flat_scratch[0:3]]
    scratch_shared = flat_scratch[3:6]
    lanes = lax.iota(jnp.int32, L)
    zero = jnp.zeros((L,), jnp.float32)
    seven = jnp.full((L,), 7, jnp.int32)

    pltpu.sync_copy(u_hbm.at[pl.ds(base, R)], uidx)
    pltpu.sync_copy(i_hbm.at[pl.ds(base, R)], iidx)
    pltpu.sync_copy(jst_hbm.at[:, pl.ds(base, R)], jidx)

    jbuf, jsem, obuf = scratch_shared

    def fire_ui(ch, b):
        ubuf, ibuf, sem = scratch[b]
        sl = pl.ds(ch * C, C)
        rv_u = uidx[sl]
        rv_i = iidx[sl]
        for l in range(L):
            r8 = lax.shift_right_logical(rv_u[l], 3) * 8
            pltpu.make_async_copy(
                tu_hbm.at[pl.ds(r8, 8), :], ubuf.at[l], sem).start()
            r8 = lax.shift_right_logical(rv_i[l], 3) * 8
            pltpu.make_async_copy(
                ti_hbm.at[pl.ds(r8, 8), :], ibuf.at[l], sem).start()

    def fire_j(ch):
        sl = pl.ds(ch * C, C)
        for jn in range(N_NEG):
            rv = jidx[jn, sl]
            for l in range(L):
                r8 = lax.shift_right_logical(rv[l], 3) * 8
                pltpu.make_async_copy(
                    ti_hbm.at[pl.ds(r8, 8), :], jbuf.at[jn, l], jsem).start()

    def drain_ui(b):
        ubuf, ibuf, sem = scratch[b]
        pltpu.make_async_copy(
            tu_hbm.at[pl.ds(0, C * 8), :], ubuf.reshape(C * 8, D), sem).wait()
        pltpu.make_async_copy(
            ti_hbm.at[pl.ds(0, C * 8), :], ibuf.reshape(C * 8, D), sem).wait()

    def drain_j():
        pltpu.make_async_copy(
            ti_hbm.at[pl.ds(0, N_NEG * C * 8), :],
            jbuf.reshape(N_NEG * C * 8, D), jsem).wait()

    def compute(ch, b):
        ubuf, ibuf, sem = scratch[b]
        uflat = ubuf.reshape(C * 8, D)
        iflat = ibuf.reshape(C * 8, D)
        jflat = jbuf.reshape(N_NEG * C * 8, D)
        sl = pl.ds(ch * C, C)
        rows_u = lanes * 8 + (uidx[sl] & seven)
        rows_i = lanes * 8 + (iidx[sl] & seven)
        rows_j = [(C * 8 * jn) + lanes * 8 + (jidx[jn, sl] & seven)
                  for jn in range(N_NEG)]

        def dbody(it, carry):
            pos, neg = carry
            for q in range(UNROLL):
                d = it * UNROLL + q
                dv = jnp.full((L,), d, jnp.int32)
                uv = plsc.load_gather(uflat, [rows_u, dv])
                iv = plsc.load_gather(iflat, [rows_i, dv])
                jsum = zero
                for jn in range(N_NEG):
                    jsum = jsum + plsc.load_gather(jflat, [rows_j[jn], dv])
                pos = pos + uv * iv
                neg = neg + uv * jsum
            return pos, neg

        pos, neg = lax.fori_loop(0, D // UNROLL, dbody, (zero, zero))
        x = pos - neg * (1.0 / N_NEG)
        obuf[:] = 1.0 / (1.0 + jnp.exp(-x))
        pltpu.sync_copy(obuf, out_hbm.at[pl.ds(base + ch * C, C)])

    def body(ch, _):
        fire_ui(ch, 0)
        fire_j(ch)
        drain_ui(0)
        drain_j()
        compute(ch, 0)
        return 0

    lax.fori_loop(0, NCHUNK, body, 0)


def _buf_scratch():
    return [
        pltpu.VMEM((C, 8, D), jnp.float32),
        pltpu.VMEM((C, 8, D), jnp.float32),
        pltpu.SemaphoreType.DMA,
    ]


_bpr = functools.partial(
    pl.kernel,
    mesh=plsc.VectorSubcoreMesh(core_axis_name="c", subcore_axis_name="s"),
    compiler_params=pltpu.CompilerParams(
        needs_layout_passes=False, use_tc_tiling_on_sc=True),
    out_type=jax.ShapeDtypeStruct((B,), jnp.float32),
    scratch_types=[
        pltpu.VMEM((R,), jnp.int32),
        pltpu.VMEM((R,), jnp.int32),
        pltpu.VMEM((N_NEG, R), jnp.int32),
    ] + _buf_scratch() + [
        pltpu.VMEM((N_NEG, C, 8, D), jnp.float32),
        pltpu.SemaphoreType.DMA,
        pltpu.VMEM((C,), jnp.float32),
    ],
)(_bpr_body)


def kernel(u, i, js, emb_user, emb_item):
    js_t = js.T
    return _bpr(u.astype(jnp.int32), i.astype(jnp.int32),
                js_t.astype(jnp.int32), emb_user, emb_item)


# speedup vs baseline: 1.3359x; 1.0728x over previous
import functools

import jax
import jax.numpy as jnp
from jax import lax
from jax.experimental import pallas as pl
from jax.experimental.pallas import tpu as pltpu
from jax.experimental.pallas import tpu_sc as plsc

B = 16384
D = 64
N_NEG = 4
NC = 2
NS = 16
L = 16
NW = NC * NS
R = B // NW
C = 16
NCHUNK = R // C
UNROLL = 4


def _bpr_body(u_hbm, i_hbm, jst_hbm, tu_hbm, ti_hbm, out_hbm,
              uidx, iidx, jidx, ubuf0, ibuf0, uisem0, ubuf1, ibuf1, uisem1,
              jbuf0, jsem0, jbuf1, jsem1, obuf):
    wid = lax.axis_index("s") * NC + lax.axis_index("c")
    base = wid * R
    ui = [(ubuf0, ibuf0, uisem0), (ubuf1, ibuf1, uisem1)]
    jb = [(jbuf0, jsem0), (jbuf1, jsem1)]
    lanes = lax.iota(jnp.int32, L)
    zero = jnp.zeros((L,), jnp.float32)
    seven = jnp.full((L,), 7, jnp.int32)

    pltpu.sync_copy(u_hbm.at[pl.ds(base, R)], uidx)
    pltpu.sync_copy(i_hbm.at[pl.ds(base, R)], iidx)
    pltpu.sync_copy(jst_hbm.at[:, pl.ds(base, R)], jidx)

    def fire_ui(ch, b):
        ubuf, ibuf, sem = ui[b]
        sl = pl.ds(ch * C, C)
        r8u = lax.shift_right_logical(uidx[sl], 3) * 8
        r8i = lax.shift_right_logical(iidx[sl], 3) * 8
        for l in range(L):
            pltpu.make_async_copy(
                tu_hbm.at[pl.ds(pl.multiple_of(r8u[l], 8), 8), :],
                ubuf.at[l], sem).start()
            pltpu.make_async_copy(
                ti_hbm.at[pl.ds(pl.multiple_of(r8i[l], 8), 8), :],
                ibuf.at[l], sem).start()

    def fire_j(ch, jn, p):
        jbuf, sem = jb[p]
        r8 = lax.shift_right_logical(jidx[jn, pl.ds(ch * C, C)], 3) * 8
        for l in range(L):
            pltpu.make_async_copy(
                ti_hbm.at[pl.ds(pl.multiple_of(r8[l], 8), 8), :],
                jbuf.at[l], sem).start()

    def drain_ui(b):
        ubuf, ibuf, sem = ui[b]
        pltpu.make_async_copy(
            tu_hbm.at[pl.ds(0, C * 8), :], ubuf.reshape(C * 8, D), sem).wait()
        pltpu.make_async_copy(
            ti_hbm.at[pl.ds(0, C * 8), :], ibuf.reshape(C * 8, D), sem).wait()

    def drain_j(p):
        jbuf, sem = jb[p]
        pltpu.make_async_copy(
            ti_hbm.at[pl.ds(0, C * 8), :], jbuf.reshape(C * 8, D), sem).wait()

    def pair_reduce(aflat, rows_a, bflat, rows_b, acc0):
        def dbody(it, acc):
            for q in range(UNROLL):
                dv = jnp.full((L,), it * UNROLL + q, jnp.int32)
                av = plsc.load_gather(aflat, [rows_a, dv])
                bv = plsc.load_gather(bflat, [rows_b, dv])
                acc = acc + av * bv
            return acc
        return lax.fori_loop(0, D // UNROLL, dbody, acc0)

    fire_ui(0, 0)
    fire_ui(1, 1)
    fire_j(0, 0, 0)

    def chunk(ch, b):
        ubuf, ibuf, _ = ui[b]
        uflat = ubuf.reshape(C * 8, D)
        iflat = ibuf.reshape(C * 8, D)
        sl = pl.ds(ch * C, C)
        rows_u = lanes * 8 + (uidx[sl] & seven)
        rows_i = lanes * 8 + (iidx[sl] & seven)

        drain_ui(b)
        pos = pair_reduce(uflat, rows_u, iflat, rows_i, zero)

        neg = zero
        for jn in range(N_NEG):
            p = jn & 1
            drain_j(p)
            if jn + 1 < N_NEG:
                fire_j(ch, jn + 1, p ^ 1)
            else:
                @pl.when(ch + 1 < NCHUNK)
                def _():
                    fire_j(ch + 1, 0, p ^ 1)
            jbuf, _ = jb[p]
            rows_j = lanes * 8 + (jidx[jn, sl] & seven)
            neg = pair_reduce(uflat, rows_u, jbuf.reshape(C * 8, D),
                              rows_j, neg)

        @pl.when(ch + 2 < NCHUNK)
        def _():
            fire_ui(ch + 2, b)

        x = pos - neg * (1.0 / N_NEG)
        obuf[:] = 1.0 / (1.0 + jnp.exp(-x))
        pltpu.sync_copy(obuf, out_hbm.at[pl.ds(base + ch * C, C)])

    def body(it, _):
        for b in range(2):
            chunk(it * 2 + b, b)
        return 0

    lax.fori_loop(0, NCHUNK // 2, body, 0)


_bpr = functools.partial(
    pl.kernel,
    mesh=plsc.VectorSubcoreMesh(core_axis_name="c", subcore_axis_name="s"),
    compiler_params=pltpu.CompilerParams(
        needs_layout_passes=False, use_tc_tiling_on_sc=True),
    out_type=jax.ShapeDtypeStruct((B,), jnp.float32),
    scratch_types=[
        pltpu.VMEM((R,), jnp.int32),
        pltpu.VMEM((R,), jnp.int32),
        pltpu.VMEM((N_NEG, R), jnp.int32),
        pltpu.VMEM((C, 8, D), jnp.float32),
        pltpu.VMEM((C, 8, D), jnp.float32),
        pltpu.SemaphoreType.DMA,
        pltpu.VMEM((C, 8, D), jnp.float32),
        pltpu.VMEM((C, 8, D), jnp.float32),
        pltpu.SemaphoreType.DMA,
        pltpu.VMEM((C, 8, D), jnp.float32),
        pltpu.SemaphoreType.DMA,
        pltpu.VMEM((C, 8, D), jnp.float32),
        pltpu.SemaphoreType.DMA,
        pltpu.VMEM((C,), jnp.float32),
    ],
)(_bpr_body)


def kernel(u, i, js, emb_user, emb_item):
    js_t = js.T
    return _bpr(u.astype(jnp.int32), i.astype(jnp.int32),
                js_t.astype(jnp.int32), emb_user, emb_item)
